# baseline (speedup 1.0000x reference)
import jax
import jax.numpy as jnp
from jax.experimental import pallas as pl

_B, _T, _D = 1, 2048, 1024
_E, _TOPK = 8, 2
_DFF = 4 * _D
_N = _B * _T

_TM = 256
_DFFT = 512
_F = _DFF // _DFFT
_NE = _E + 1


def _router_body(x_ref, wr_ref, gates_ref):
    x = x_ref[...]
    wr = wr_ref[...]
    logits = jax.lax.dot_general(
        x, wr, (((1,), (1,)), ((), ())),
        preferred_element_type=jnp.float32)
    p = jax.nn.sigmoid(logits)
    col = jax.lax.broadcasted_iota(jnp.int32, p.shape, 1)
    m1 = jnp.max(p, axis=1, keepdims=True)
    a1 = jnp.min(jnp.where(p == m1, col, _E + 1), axis=1, keepdims=True)
    mask1 = col == a1
    p2 = jnp.where(mask1, -1.0, p)
    m2 = jnp.max(p2, axis=1, keepdims=True)
    a2 = jnp.min(jnp.where(p2 == m2, col, _E + 1), axis=1, keepdims=True)
    mask2 = col == a2
    denom = m1 + m2 + 1e-6
    gates_ref[...] = jnp.where(mask1, m1 / denom,
                               jnp.where(mask2, m2 / denom, 0.0))


def _router(xf, Wr):
    return pl.pallas_call(
        _router_body,
        out_shape=jax.ShapeDtypeStruct((_N, _E), jnp.float32),
    )(xf, Wr)


def _moe_body(x_ref, wu_ref, wd_ref, g_ref, out_ref):
    e = pl.program_id(1)
    f = pl.program_id(2)
    x = x_ref[...]
    wu = wu_ref[0]
    wd = wd_ref[0]
    h = jax.lax.dot_general(
        x, wu, (((1,), (1,)), ((), ())),
        preferred_element_type=jnp.float32)
    h = jnp.square(jnp.maximum(h, 0.0)).astype(jnp.bfloat16)
    y = jax.lax.dot_general(
        h, wd, (((1,), (1,)), ((), ())),
        preferred_element_type=jnp.float32)
    gall = g_ref[...]
    col = jax.lax.broadcasted_iota(jnp.int32, gall.shape, 1)
    g = jnp.sum(jnp.where(col == e, gall, 0.0), axis=1)
    contrib = g[:, None] * y

    @pl.when(jnp.logical_and(e == 0, f == 0))
    def _():
        out_ref[...] = contrib

    @pl.when(jnp.logical_not(jnp.logical_and(e == 0, f == 0)))
    def _():
        out_ref[...] += contrib


def _moe(xb, Wu_all, Wd_all, gates):
    grid = (_N // _TM, _NE, _F)
    return pl.pallas_call(
        _moe_body,
        grid=grid,
        in_specs=[
            pl.BlockSpec((_TM, _D), lambda t, e, f: (t, 0)),
            pl.BlockSpec((1, _DFFT, _D), lambda t, e, f: (e, f, 0)),
            pl.BlockSpec((1, _D, _DFFT), lambda t, e, f: (e, 0, f)),
            pl.BlockSpec((_TM, _NE), lambda t, e, f: (t, 0)),
        ],
        out_specs=pl.BlockSpec((_TM, _D), lambda t, e, f: (t, 0)),
        out_shape=jax.ShapeDtypeStruct((_N, _D), jnp.float32),
    )(xb, Wu_all, Wd_all, gates)


def kernel(x, Wr, Wu, Wd, Ws1, Ws2):
    xf = x.reshape(_N, _D)
    gates_e = _router(xf, Wr)
    gates = jnp.concatenate(
        [gates_e, jnp.ones((_N, 1), jnp.float32)], axis=1)
    xb = xf.astype(jnp.bfloat16)
    Wu_all = jnp.concatenate([Wu, Ws1[None]], axis=0).astype(jnp.bfloat16)
    Wd_all = jnp.concatenate([Wd, Ws2[None]], axis=0).astype(jnp.bfloat16)
    out = _moe(xb, Wu_all, Wd_all, gates)
    return out.reshape(_B, _T, _D)


# speedup vs baseline: 1.4969x; 1.4969x over previous
import functools

import jax
import jax.numpy as jnp
from jax import lax
from jax.experimental import pallas as pl
from jax.experimental.pallas import tpu as pltpu
from jax.experimental.pallas import tpu_sc as plsc

_B, _T, _D = 1, 2048, 1024
_E, _TOPK = 8, 2
_DFF = 4 * _D
_N = _B * _T
_A = _N * _TOPK

_TM = 256
_APAD = _A + _E * _TM
_R = _N + _APAD
_G = _R // _TM


def _router_body(x_ref, wr_ref, idx_ref, w_ref):
    x = x_ref[...]
    wr = wr_ref[...]
    logits = jax.lax.dot_general(
        x, wr, (((1,), (1,)), ((), ())),
        preferred_element_type=jnp.float32)
    p = jax.nn.sigmoid(logits)
    col = jax.lax.broadcasted_iota(jnp.int32, p.shape, 1)
    m1 = jnp.max(p, axis=1, keepdims=True)
    a1 = jnp.min(jnp.where(p == m1, col, _E + 1), axis=1, keepdims=True)
    p2 = jnp.where(col == a1, -1.0, p)
    m2 = jnp.max(p2, axis=1, keepdims=True)
    a2 = jnp.min(jnp.where(p2 == m2, col, _E + 1), axis=1, keepdims=True)
    denom = m1 + m2 + 1e-6
    idx_ref[...] = jnp.concatenate([a1, a2], axis=1)
    w_ref[...] = jnp.concatenate([m1 / denom, m2 / denom], axis=1)


def _router(xf, Wr):
    return pl.pallas_call(
        _router_body,
        out_shape=[
            jax.ShapeDtypeStruct((_N, _TOPK), jnp.int32),
            jax.ShapeDtypeStruct((_N, _TOPK), jnp.float32),
        ],
    )(xf, Wr)


def _gather_rows(xf, src_idx):
    info = plsc.get_sparse_core_info()
    nw = info.num_cores * info.num_subcores
    per_w = _R // nw
    ch = 64
    mesh = plsc.VectorSubcoreMesh(core_axis_name="c", subcore_axis_name="s")

    @functools.partial(
        pl.kernel, mesh=mesh,
        out_type=jax.ShapeDtypeStruct((_R, _D), jnp.float32),
        scratch_types=[
            pltpu.VMEM((ch,), jnp.int32),
            pltpu.VMEM((ch, _D), jnp.float32),
            pltpu.SemaphoreType.DMA,
        ],
    )
    def k(x_hbm, idx_hbm, out_hbm, idx_v, rows_v, sem):
        wid = lax.axis_index("s") * info.num_cores + lax.axis_index("c")
        base = wid * per_w
        for c in range(per_w // ch):
            off = base + c * ch
            pltpu.sync_copy(idx_hbm.at[pl.ds(off, ch)], idx_v)
            pltpu.async_copy(x_hbm.at[idx_v], rows_v, sem).wait()
            pltpu.sync_copy(rows_v, out_hbm.at[pl.ds(off, ch)])

    return k(xf, src_idx)


def _ffn_body(eot_ref, x_ref, wu_ref, wd_ref, g_ref, y_ref):
    del eot_ref
    x = x_ref[...].astype(jnp.bfloat16)
    h = jax.lax.dot_general(
        x, wu_ref[0], (((1,), (1,)), ((), ())),
        preferred_element_type=jnp.float32)
    h = jnp.square(jnp.maximum(h, 0.0)).astype(jnp.bfloat16)
    y = jax.lax.dot_general(
        h, wd_ref[0], (((1,), (1,)), ((), ())),
        preferred_element_type=jnp.float32)
    g = g_ref[0, 0, :]
    y_ref[...] = g[:, None] * y


def _ffn(x_all, Wu_all, Wd_all, gates_all, eot):
    grid_spec = pltpu.PrefetchScalarGridSpec(
        num_scalar_prefetch=1,
        grid=(_G,),
        in_specs=[
            pl.BlockSpec((_TM, _D), lambda t, eot: (t, 0)),
            pl.BlockSpec((1, _DFF, _D), lambda t, eot: (eot[t], 0, 0)),
            pl.BlockSpec((1, _D, _DFF), lambda t, eot: (eot[t], 0, 0)),
            pl.BlockSpec((1, 1, _TM), lambda t, eot: (t, 0, 0)),
        ],
        out_specs=pl.BlockSpec((_TM, _D), lambda t, eot: (t, 0)),
    )
    return pl.pallas_call(
        _ffn_body,
        grid_spec=grid_spec,
        out_shape=jax.ShapeDtypeStruct((_R, _D), jnp.float32),
    )(eot, x_all, Wu_all, Wd_all, gates_all)


def _combine_rows(y_all, inv0, inv1):
    info = plsc.get_sparse_core_info()
    nw = info.num_cores * info.num_subcores
    per_w = _N // nw
    ch = 16
    nvec = _D // 16
    mesh = plsc.VectorSubcoreMesh(core_axis_name="c", subcore_axis_name="s")

    @functools.partial(
        pl.kernel, mesh=mesh,
        out_type=jax.ShapeDtypeStruct((_N, _D), jnp.float32),
        scratch_types=[
            pltpu.VMEM((ch,), jnp.int32),
            pltpu.VMEM((ch,), jnp.int32),
            pltpu.VMEM((ch, _D), jnp.float32),
            pltpu.VMEM((ch, _D), jnp.float32),
            pltpu.VMEM((ch, _D), jnp.float32),
            pltpu.SemaphoreType.DMA,
        ],
    )
    def k(y_hbm, i0_hbm, i1_hbm, out_hbm, i0_v, i1_v, bs, b0, b1, sem):
        wid = lax.axis_index("s") * info.num_cores + lax.axis_index("c")
        base = wid * per_w
        for c in range(per_w // ch):
            off = base + c * ch
            pltpu.sync_copy(i0_hbm.at[pl.ds(off, ch)], i0_v)
            pltpu.sync_copy(i1_hbm.at[pl.ds(off, ch)], i1_v)
            pltpu.sync_copy(y_hbm.at[pl.ds(off, ch)], bs)
            pltpu.async_copy(y_hbm.at[i0_v], b0, sem).wait()
            pltpu.async_copy(y_hbm.at[i1_v], b1, sem).wait()

            def col(j, _):
                for r in range(ch):
                    sl = (r, pl.ds(j * 16, 16))
                    bs[sl] = bs[sl] + b0[sl] + b1[sl]
                return 0

            lax.fori_loop(0, nvec, col, 0)
            pltpu.sync_copy(bs, out_hbm.at[pl.ds(off, ch)])

    return k(y_all, inv0, inv1)


def kernel(x, Wr, Wu, Wd, Ws1, Ws2):
    xf = x.reshape(_N, _D)
    idx2, w2 = _router(xf, Wr)

    e_flat = idx2.reshape(-1)
    order = jnp.argsort(e_flat, stable=True).astype(jnp.int32)
    e_sorted = e_flat[order]
    counts = jnp.zeros(_E, jnp.int32).at[e_flat].add(1)
    offsets = jnp.cumsum(counts) - counts
    tiles_e = (counts + _TM - 1) // _TM
    tcum = jnp.cumsum(tiles_e)
    pad_off = _TM * (tcum - tiles_e)
    rank = jnp.arange(_A, dtype=jnp.int32) - offsets[e_sorted]
    dest = pad_off[e_sorted] + rank
    token_src = jnp.zeros(_APAD, jnp.int32).at[dest].set(order // _TOPK)
    src_idx = jnp.concatenate(
        [jnp.arange(_N, dtype=jnp.int32), token_src])
    gates_pad = jnp.zeros(_APAD, jnp.float32).at[dest].set(w2.reshape(-1)[order])
    gates_all = jnp.concatenate(
        [jnp.ones(_N, jnp.float32), gates_pad]).reshape(_G, 1, _TM)
    assign_eot = jnp.clip(
        jnp.searchsorted(tcum, jnp.arange(_APAD // _TM), side="right"),
        0, _E - 1).astype(jnp.int32)
    eot = jnp.concatenate(
        [jnp.full(_N // _TM, _E, jnp.int32), assign_eot])
    inv_flat = jnp.zeros(_A, jnp.int32).at[order].set(dest + _N)
    inv2 = inv_flat.reshape(_N, _TOPK)
    inv0 = inv2[:, 0]
    inv1 = inv2[:, 1]

    x_all = _gather_rows(xf, src_idx)
    Wu_all = jnp.concatenate([Wu, Ws1[None]], axis=0).astype(jnp.bfloat16)
    Wd_all = jnp.concatenate([Wd, Ws2[None]], axis=0).astype(jnp.bfloat16)
    y_all = _ffn(x_all, Wu_all, Wd_all, gates_all, eot)
    out = _combine_rows(y_all, inv0, inv1)
    return out.reshape(_B, _T, _D)


# speedup vs baseline: 1.8342x; 1.2253x over previous
import functools

import jax
import jax.numpy as jnp
from jax import lax
from jax.experimental import pallas as pl
from jax.experimental.pallas import tpu as pltpu
from jax.experimental.pallas import tpu_sc as plsc

_B, _T, _D = 1, 2048, 1024
_E, _TOPK = 8, 2
_DFF = 4 * _D
_N = _B * _T
_A = _N * _TOPK

_TM = 256
_APAD = _A + _E * _TM
_GA = _APAD // _TM
_GS = _N // _TM


def _router_body(x_ref, wr_ref, idx_ref, w_ref):
    x = x_ref[...]
    wr = wr_ref[...]
    logits = jax.lax.dot_general(
        x, wr, (((1,), (1,)), ((), ())),
        preferred_element_type=jnp.float32)
    p = jax.nn.sigmoid(logits)
    col = jax.lax.broadcasted_iota(jnp.int32, p.shape, 1)
    m1 = jnp.max(p, axis=1, keepdims=True)
    a1 = jnp.min(jnp.where(p == m1, col, _E + 1), axis=1, keepdims=True)
    p2 = jnp.where(col == a1, -1.0, p)
    m2 = jnp.max(p2, axis=1, keepdims=True)
    a2 = jnp.min(jnp.where(p2 == m2, col, _E + 1), axis=1, keepdims=True)
    denom = m1 + m2 + 1e-6
    idx_ref[...] = jnp.concatenate([a1, a2], axis=1)
    w_ref[...] = jnp.concatenate([m1 / denom, m2 / denom], axis=1)


def _router(xf, Wr):
    return pl.pallas_call(
        _router_body,
        out_shape=[
            jax.ShapeDtypeStruct((_N, _TOPK), jnp.int32),
            jax.ShapeDtypeStruct((_N, _TOPK), jnp.float32),
        ],
    )(xf, Wr)


def _gather_rows(xf, src_idx):
    info = plsc.get_sparse_core_info()
    nw = info.num_cores * info.num_subcores
    per_w = _APAD // nw
    ch = 32
    nch = per_w // ch
    mesh = plsc.VectorSubcoreMesh(core_axis_name="c", subcore_axis_name="s")

    nbuf = 3

    @functools.partial(
        pl.kernel, mesh=mesh,
        out_type=jax.ShapeDtypeStruct((_APAD, _D), jnp.float32),
        scratch_types=(
            [pltpu.VMEM((per_w,), jnp.int32)]
            + [pltpu.VMEM((ch, _D), jnp.float32) for _ in range(nbuf)]
            + [pltpu.SemaphoreType.DMA for _ in range(2 * nbuf)]
        ),
    )
    def k(x_hbm, idx_hbm, out_hbm, idx_v, *rest):
        bufs = rest[:nbuf]
        gsems = rest[nbuf:2 * nbuf]
        ssems = rest[2 * nbuf:3 * nbuf]
        wid = lax.axis_index("s") * info.num_cores + lax.axis_index("c")
        base = wid * per_w
        pltpu.sync_copy(idx_hbm.at[pl.ds(base, per_w)], idx_v)

        def gather(c):
            b = c % nbuf
            return pltpu.async_copy(
                x_hbm.at[idx_v.at[pl.ds(c * ch, ch)]], bufs[b], gsems[b])

        gathers = {c: gather(c) for c in range(min(nbuf, nch))}
        stores = {}
        for c in range(nch):
            b = c % nbuf
            gathers[c].wait()
            stores[c] = pltpu.async_copy(
                bufs[b], out_hbm.at[pl.ds(base + c * ch, ch)], ssems[b])
            if c + nbuf < nch:
                stores[c].wait()
                gathers[c + nbuf] = gather(c + nbuf)
        for c in range(max(0, nch - nbuf), nch):
            stores[c].wait()

    return k(xf, src_idx)


def _shared_body(x_ref, wu_ref, wd_ref, y_ref):
    h = jax.lax.dot_general(
        x_ref[...], wu_ref[...], (((1,), (1,)), ((), ())),
        preferred_element_type=jnp.float32)
    h = jnp.square(jnp.maximum(h, 0.0)).astype(jnp.bfloat16)
    y_ref[...] = jax.lax.dot_general(
        h, wd_ref[...], (((1,), (1,)), ((), ())),
        preferred_element_type=jnp.float32)


def _shared_ffn(xb, Ws1b, Ws2b):
    return pl.pallas_call(
        _shared_body,
        grid=(_GS,),
        in_specs=[
            pl.BlockSpec((_TM, _D), lambda t: (t, 0)),
            pl.BlockSpec((_DFF, _D), lambda t: (0, 0)),
            pl.BlockSpec((_D, _DFF), lambda t: (0, 0)),
        ],
        out_specs=pl.BlockSpec((_TM, _D), lambda t: (t, 0)),
        out_shape=jax.ShapeDtypeStruct((_N, _D), jnp.float32),
    )(xb, Ws1b, Ws2b)


def _routed_body(eot_ref, x_ref, wu_ref, wd_ref, g_ref, y_ref):
    del eot_ref
    h = jax.lax.dot_general(
        x_ref[...].astype(jnp.bfloat16), wu_ref[0], (((1,), (1,)), ((), ())),
        preferred_element_type=jnp.float32)
    h = jnp.square(jnp.maximum(h, 0.0)).astype(jnp.bfloat16)
    y = jax.lax.dot_general(
        h, wd_ref[0], (((1,), (1,)), ((), ())),
        preferred_element_type=jnp.float32)
    g = g_ref[0, 0, :]
    y_ref[...] = g[:, None] * y


def _routed_ffn(x_routed, Wub, Wdb, gates, eot):
    grid_spec = pltpu.PrefetchScalarGridSpec(
        num_scalar_prefetch=1,
        grid=(_GA,),
        in_specs=[
            pl.BlockSpec((_TM, _D), lambda t, eot: (t, 0)),
            pl.BlockSpec((1, _DFF, _D), lambda t, eot: (eot[t], 0, 0)),
            pl.BlockSpec((1, _D, _DFF), lambda t, eot: (eot[t], 0, 0)),
            pl.BlockSpec((1, 1, _TM), lambda t, eot: (t, 0, 0)),
        ],
        out_specs=pl.BlockSpec((_TM, _D), lambda t, eot: (t, 0)),
    )
    return pl.pallas_call(
        _routed_body,
        grid_spec=grid_spec,
        out_shape=jax.ShapeDtypeStruct((_APAD, _D), jnp.float32),
    )(eot, x_routed, Wub, Wdb, gates)


def _combine_rows(y_shared, y_routed, inv0, inv1):
    info = plsc.get_sparse_core_info()
    nw = info.num_cores * info.num_subcores
    per_w = _N // nw
    ch = 16
    nvec = _D // 16
    mesh = plsc.VectorSubcoreMesh(core_axis_name="c", subcore_axis_name="s")

    @functools.partial(
        pl.kernel, mesh=mesh,
        out_type=jax.ShapeDtypeStruct((_N, _D), jnp.float32),
        scratch_types=[
            pltpu.VMEM((ch,), jnp.int32),
            pltpu.VMEM((ch,), jnp.int32),
            pltpu.VMEM((ch, _D), jnp.float32),
            pltpu.VMEM((ch, _D), jnp.float32),
            pltpu.VMEM((ch, _D), jnp.float32),
            pltpu.SemaphoreType.DMA,
        ],
    )
    def k(ys_hbm, yr_hbm, i0_hbm, i1_hbm, out_hbm, i0_v, i1_v, bs, b0, b1, sem):
        wid = lax.axis_index("s") * info.num_cores + lax.axis_index("c")
        base = wid * per_w
        for c in range(per_w // ch):
            off = base + c * ch
            pltpu.sync_copy(i0_hbm.at[pl.ds(off, ch)], i0_v)
            pltpu.sync_copy(i1_hbm.at[pl.ds(off, ch)], i1_v)
            pltpu.sync_copy(ys_hbm.at[pl.ds(off, ch)], bs)
            pltpu.async_copy(yr_hbm.at[i0_v], b0, sem).wait()
            pltpu.async_copy(yr_hbm.at[i1_v], b1, sem).wait()

            def col(j, _):
                for r in range(ch):
                    sl = (r, pl.ds(j * 16, 16))
                    bs[sl] = bs[sl] + b0[sl] + b1[sl]
                return 0

            lax.fori_loop(0, nvec, col, 0)
            pltpu.sync_copy(bs, out_hbm.at[pl.ds(off, ch)])

    return k(y_shared, y_routed, inv0, inv1)


def kernel(x, Wr, Wu, Wd, Ws1, Ws2):
    xf = x.reshape(_N, _D)
    idx2, w2 = _router(xf, Wr)

    e_flat = idx2.reshape(-1)
    oh = (e_flat[:, None] == jnp.arange(_E, dtype=jnp.int32)[None, :])
    cnt_cum = jnp.cumsum(oh.astype(jnp.int32), axis=0)
    counts = cnt_cum[-1]
    rank = jnp.take_along_axis(cnt_cum, e_flat[:, None], axis=1)[:, 0] - 1
    tiles_e = (counts + _TM - 1) // _TM
    tcum = jnp.cumsum(tiles_e)
    pad_off = _TM * (tcum - tiles_e)
    dest = pad_off[e_flat] + rank
    ar = jnp.arange(_A, dtype=jnp.int32)
    src_idx = jnp.zeros(_APAD, jnp.int32).at[dest].set(ar // _TOPK)
    gates = jnp.zeros(_APAD, jnp.float32).at[dest].set(
        w2.reshape(-1)).reshape(_GA, 1, _TM)
    eot = jnp.clip(
        jnp.searchsorted(tcum, jnp.arange(_GA), side="right"),
        0, _E - 1).astype(jnp.int32)
    inv2 = dest.reshape(_N, _TOPK)
    inv0 = inv2[:, 0]
    inv1 = inv2[:, 1]

    xb = xf.astype(jnp.bfloat16)
    y_shared = _shared_ffn(xb, Ws1.astype(jnp.bfloat16), Ws2.astype(jnp.bfloat16))
    x_routed = _gather_rows(xf, src_idx)
    y_routed = _routed_ffn(x_routed, Wu.astype(jnp.bfloat16),
                           Wd.astype(jnp.bfloat16), gates, eot)
    out = _combine_rows(y_shared, y_routed, inv0, inv1)
    return out.reshape(_B, _T, _D)


# speedup vs baseline: 2.2486x; 1.2259x over previous
import functools

import jax
import jax.numpy as jnp
from jax import lax
from jax.experimental import pallas as pl
from jax.experimental.pallas import tpu as pltpu
from jax.experimental.pallas import tpu_sc as plsc

_B, _T, _D = 1, 2048, 1024
_E, _TOPK = 8, 2
_DFF = 4 * _D
_N = _B * _T
_A = _N * _TOPK

_TM = 256
_APAD = _A + _E * _TM
_GA = _APAD // _TM
_GS = _N // _TM


def _router_body(x_ref, wr_ref, idx_ref, w_ref):
    x = x_ref[...]
    wr = wr_ref[...]
    logits = jax.lax.dot_general(
        x, wr, (((1,), (1,)), ((), ())),
        preferred_element_type=jnp.float32)
    p = jax.nn.sigmoid(logits)
    col = jax.lax.broadcasted_iota(jnp.int32, p.shape, 1)
    m1 = jnp.max(p, axis=1, keepdims=True)
    a1 = jnp.min(jnp.where(p == m1, col, _E + 1), axis=1, keepdims=True)
    p2 = jnp.where(col == a1, -1.0, p)
    m2 = jnp.max(p2, axis=1, keepdims=True)
    a2 = jnp.min(jnp.where(p2 == m2, col, _E + 1), axis=1, keepdims=True)
    denom = m1 + m2 + 1e-6
    idx_ref[...] = jnp.concatenate([a1, a2], axis=1)
    w_ref[...] = jnp.concatenate([m1 / denom, m2 / denom], axis=1)


def _router(xf, Wr):
    return pl.pallas_call(
        _router_body,
        out_shape=[
            jax.ShapeDtypeStruct((_N, _TOPK), jnp.int32),
            jax.ShapeDtypeStruct((_N, _TOPK), jnp.float32),
        ],
    )(xf, Wr)


def _gather_rows(xf, src_idx):
    info = plsc.get_sparse_core_info()
    nw = info.num_cores * info.num_subcores
    per_w = _APAD // nw
    ch = 32
    nch = per_w // ch
    mesh = plsc.VectorSubcoreMesh(core_axis_name="c", subcore_axis_name="s")

    nbuf = 3

    @functools.partial(
        pl.kernel, mesh=mesh,
        out_type=jax.ShapeDtypeStruct((_APAD, _D), jnp.float32),
        scratch_types=(
            [pltpu.VMEM((per_w,), jnp.int32)]
            + [pltpu.VMEM((ch, _D), jnp.float32) for _ in range(nbuf)]
            + [pltpu.SemaphoreType.DMA for _ in range(2 * nbuf)]
        ),
    )
    def k(x_hbm, idx_hbm, out_hbm, idx_v, *rest):
        bufs = rest[:nbuf]
        gsems = rest[nbuf:2 * nbuf]
        ssems = rest[2 * nbuf:3 * nbuf]
        wid = lax.axis_index("s") * info.num_cores + lax.axis_index("c")
        base = wid * per_w
        pltpu.sync_copy(idx_hbm.at[pl.ds(base, per_w)], idx_v)

        def gather(c):
            b = c % nbuf
            return pltpu.async_copy(
                x_hbm.at[idx_v.at[pl.ds(c * ch, ch)]], bufs[b], gsems[b])

        gathers = {c: gather(c) for c in range(min(nbuf, nch))}
        stores = {}
        for c in range(nch):
            b = c % nbuf
            gathers[c].wait()
            stores[c] = pltpu.async_copy(
                bufs[b], out_hbm.at[pl.ds(base + c * ch, ch)], ssems[b])
            if c + nbuf < nch:
                stores[c].wait()
                gathers[c + nbuf] = gather(c + nbuf)
        for c in range(max(0, nch - nbuf), nch):
            stores[c].wait()

    return k(xf, src_idx)


def _shared_body(x_ref, wu_ref, wd_ref, y_ref):
    h = jax.lax.dot_general(
        x_ref[...], wu_ref[...], (((1,), (1,)), ((), ())),
        preferred_element_type=jnp.float32)
    h = jnp.square(jnp.maximum(h, 0.0)).astype(jnp.bfloat16)
    y_ref[...] = jax.lax.dot_general(
        h, wd_ref[...], (((1,), (1,)), ((), ())),
        preferred_element_type=jnp.float32)


def _shared_ffn(xb, Ws1b, Ws2b):
    return pl.pallas_call(
        _shared_body,
        grid=(_GS,),
        in_specs=[
            pl.BlockSpec((_TM, _D), lambda t: (t, 0)),
            pl.BlockSpec((_DFF, _D), lambda t: (0, 0)),
            pl.BlockSpec((_D, _DFF), lambda t: (0, 0)),
        ],
        out_specs=pl.BlockSpec((_TM, _D), lambda t: (t, 0)),
        out_shape=jax.ShapeDtypeStruct((_N, _D), jnp.float32),
    )(xb, Ws1b, Ws2b)


def _routed_body(eot_ref, x_ref, wu_ref, wd_ref, g_ref, y_ref):
    del eot_ref
    h = jax.lax.dot_general(
        x_ref[...].astype(jnp.bfloat16), wu_ref[0], (((1,), (1,)), ((), ())),
        preferred_element_type=jnp.float32)
    h = jnp.square(jnp.maximum(h, 0.0)).astype(jnp.bfloat16)
    y = jax.lax.dot_general(
        h, wd_ref[0], (((1,), (1,)), ((), ())),
        preferred_element_type=jnp.float32)
    g = g_ref[0, 0, :]
    y_ref[...] = g[:, None] * y


def _routed_ffn(x_routed, Wub, Wdb, gates, eot):
    grid_spec = pltpu.PrefetchScalarGridSpec(
        num_scalar_prefetch=1,
        grid=(_GA,),
        in_specs=[
            pl.BlockSpec((_TM, _D), lambda t, eot: (t, 0)),
            pl.BlockSpec((1, _DFF, _D), lambda t, eot: (eot[t], 0, 0)),
            pl.BlockSpec((1, _D, _DFF), lambda t, eot: (eot[t], 0, 0)),
            pl.BlockSpec((1, 1, _TM), lambda t, eot: (t, 0, 0)),
        ],
        out_specs=pl.BlockSpec((_TM, _D), lambda t, eot: (t, 0)),
    )
    return pl.pallas_call(
        _routed_body,
        grid_spec=grid_spec,
        out_shape=jax.ShapeDtypeStruct((_APAD, _D), jnp.float32),
    )(eot, x_routed, Wub, Wdb, gates)


def _combine_rows(y_shared, y_routed, inv0, inv1):
    info = plsc.get_sparse_core_info()
    nw = info.num_cores * info.num_subcores
    per_w = _N // nw
    ch = 16
    nvec = _D // 16
    mesh = plsc.VectorSubcoreMesh(core_axis_name="c", subcore_axis_name="s")

    @functools.partial(
        pl.kernel, mesh=mesh,
        out_type=jax.ShapeDtypeStruct((_N, _D), jnp.float32),
        scratch_types=[
            pltpu.VMEM((ch,), jnp.int32),
            pltpu.VMEM((ch,), jnp.int32),
            pltpu.VMEM((ch, _D), jnp.float32),
            pltpu.VMEM((ch, _D), jnp.float32),
            pltpu.VMEM((ch, _D), jnp.float32),
            pltpu.SemaphoreType.DMA,
        ],
    )
    def k(ys_hbm, yr_hbm, i0_hbm, i1_hbm, out_hbm, i0_v, i1_v, bs, b0, b1, sem):
        wid = lax.axis_index("s") * info.num_cores + lax.axis_index("c")
        base = wid * per_w
        for c in range(per_w // ch):
            off = base + c * ch
            pltpu.sync_copy(i0_hbm.at[pl.ds(off, ch)], i0_v)
            pltpu.sync_copy(i1_hbm.at[pl.ds(off, ch)], i1_v)
            pltpu.sync_copy(ys_hbm.at[pl.ds(off, ch)], bs)
            pltpu.async_copy(yr_hbm.at[i0_v], b0, sem).wait()
            pltpu.async_copy(yr_hbm.at[i1_v], b1, sem).wait()

            def col(j, _):
                for r in range(ch):
                    sl = (r, pl.ds(j * 16, 16))
                    bs[sl] = bs[sl] + b0[sl] + b1[sl]
                return 0

            lax.fori_loop(0, nvec, col, 0)
            pltpu.sync_copy(bs, out_hbm.at[pl.ds(off, ch)])

    return k(y_shared, y_routed, inv0, inv1)


def kernel(x, Wr, Wu, Wd, Ws1, Ws2):
    xf = x.reshape(_N, _D)
    idx2, w2 = _router(xf, Wr)

    e_flat = idx2.reshape(-1)
    oh = (e_flat[:, None] == jnp.arange(_E, dtype=jnp.int32)[None, :])
    cnt_cum = jnp.cumsum(oh.astype(jnp.int32), axis=0)
    counts = cnt_cum[-1]
    rank = jnp.take_along_axis(cnt_cum, e_flat[:, None], axis=1)[:, 0] - 1
    tiles_e = (counts + _TM - 1) // _TM
    tcum = jnp.cumsum(tiles_e)
    pad_off = _TM * (tcum - tiles_e)
    dest = pad_off[e_flat] + rank
    ar = jnp.arange(_A, dtype=jnp.int32)
    spread = jnp.arange(_APAD, dtype=jnp.int32) % _N
    src_idx = spread.at[dest].set(ar // _TOPK)
    gates = jnp.zeros(_APAD, jnp.float32).at[dest].set(
        w2.reshape(-1)).reshape(_GA, 1, _TM)
    eot = jnp.clip(
        jnp.searchsorted(tcum, jnp.arange(_GA), side="right"),
        0, _E - 1).astype(jnp.int32)
    inv2 = dest.reshape(_N, _TOPK)
    inv0 = inv2[:, 0]
    inv1 = inv2[:, 1]

    xb = xf.astype(jnp.bfloat16)
    y_shared = _shared_ffn(xb, Ws1.astype(jnp.bfloat16), Ws2.astype(jnp.bfloat16))
    x_routed = _gather_rows(xf, src_idx)
    y_routed = _routed_ffn(x_routed, Wu.astype(jnp.bfloat16),
                           Wd.astype(jnp.bfloat16), gates, eot)
    out = _combine_rows(y_shared, y_routed, inv0, inv1)
    return out.reshape(_B, _T, _D)
